# baseline (speedup 1.0000x reference)
import functools

import jax
import jax.numpy as jnp
from jax import lax
from jax.experimental import pallas as pl
from jax.experimental.pallas import tpu as pltpu
from jax.experimental.pallas import tpu_sc as plsc

_B = 16384
_D = 64
_LANES = 16
_NU = 1000000
_C = 128


def _gmf_kernel(b_per_w, user_idx, item_idx, user_table,
                item_table, w_vec, bias, out_hbm,
                idx_u_v, idx_i_v, blk_u, blk_i, rows_u, rows_i,
                w_v, b_v, out_v, sem0, sem1):
    n_chunks = b_per_w // _C
    wid = lax.axis_index("s") * 2 + lax.axis_index("c")
    base = wid * b_per_w

    for j in range(n_chunks):
        pltpu.sync_copy(user_idx.at[pl.ds(base + j * _C, _C)], idx_u_v.at[j])
        pltpu.sync_copy(item_idx.at[pl.ds(base + j * _C, _C)], idx_i_v.at[j])
    pltpu.sync_copy(w_vec, w_v)
    pltpu.sync_copy(bias, b_v)

    for j in range(n_chunks):
        for g in range(_C // _LANES):
            u = idx_u_v[j, pl.ds(g * _LANES, _LANES)]
            i = idx_i_v[j, pl.ds(g * _LANES, _LANES)]
            blk_u[j, pl.ds(g * _LANES, _LANES)] = lax.shift_right_logical(
                u, 1)
            blk_i[j, pl.ds(g * _LANES, _LANES)] = lax.shift_right_logical(
                i, 1)

    sems = [sem0, sem1]

    def fire_chunk(c, slot):
        pltpu.async_copy(user_table.at[blk_u.at[c]], rows_u.at[slot],
                         sems[slot])
        pltpu.async_copy(item_table.at[blk_i.at[c]], rows_i.at[slot],
                         sems[slot])

    def drain(slot):
        pltpu.make_async_copy(user_table.at[pl.ds(0, _C)],
                              rows_u.at[slot], sems[slot]).wait()
        pltpu.make_async_copy(item_table.at[pl.ds(0, _C)],
                              rows_i.at[slot], sems[slot]).wait()

    w_chunks = [w_v[pl.ds(k * _LANES, _LANES)] for k in range(_D // _LANES)]
    lane = lax.iota(jnp.int32, _LANES)
    bias_vec = b_v[...]

    def compute_chunk(c, slot):
        for g in range(_C // _LANES):
            pu = idx_u_v[c, pl.ds(g * _LANES, _LANES)] & 1
            pi = idx_i_v[c, pl.ds(g * _LANES, _LANES)] & 1
            res = jnp.zeros((_LANES,), jnp.float32)
            for e in range(_LANES):
                el = g * _LANES + e
                hu = pu[e] * _D
                hi = pi[e] * _D
                acc = (rows_u[slot, el, pl.ds(hu, _LANES)]
                       * rows_i[slot, el, pl.ds(hi, _LANES)] * w_chunks[0])
                for k in range(1, _D // _LANES):
                    acc = acc + (rows_u[slot, el, pl.ds(hu + k * _LANES,
                                                        _LANES)]
                                 * rows_i[slot, el, pl.ds(hi + k * _LANES,
                                                          _LANES)]
                                 * w_chunks[k])
                res = jnp.where(lane == e, jnp.sum(acc), res)
            x = res + bias_vec
            out_v[pl.ds(c * _C + g * _LANES, _LANES)] = 1.0 / (1.0 +
                                                               jnp.exp(-x))

    fire_chunk(0, 0)

    def pipe_body(h, carry):
        c = h * 2
        fire_chunk(c + 1, 1)
        drain(0)
        compute_chunk(c, 0)

        @pl.when(c + 2 < n_chunks)
        def _():
            fire_chunk(c + 2, 0)

        drain(1)
        compute_chunk(c + 1, 1)
        return carry

    lax.fori_loop(0, n_chunks // 2, pipe_body, 0)

    pltpu.sync_copy(out_v, out_hbm.at[pl.ds(base, b_per_w)])


def kernel(user_input, item_input, user_table, item_table, W, b):
    info = plsc.get_sparse_core_info()
    num_workers = info.num_cores * info.num_subcores
    b_per_w = _B // num_workers
    n_chunks = b_per_w // _C

    mesh = plsc.VectorSubcoreMesh(core_axis_name="c", subcore_axis_name="s")
    run = pl.kernel(
        functools.partial(_gmf_kernel, b_per_w),
        mesh=mesh,
        compiler_params=pltpu.CompilerParams(needs_layout_passes=False),
        out_type=jax.ShapeDtypeStruct((_B,), jnp.float32),
        scratch_types=[
            pltpu.VMEM((n_chunks, _C), jnp.int32),
            pltpu.VMEM((n_chunks, _C), jnp.int32),
            pltpu.VMEM((n_chunks, _C), jnp.int32),
            pltpu.VMEM((n_chunks, _C), jnp.int32),
            pltpu.VMEM((2, _C, 2 * _D), jnp.float32),
            pltpu.VMEM((2, _C, 2 * _D), jnp.float32),
            pltpu.VMEM((_D,), jnp.float32),
            pltpu.VMEM((_LANES,), jnp.float32),
            pltpu.VMEM((b_per_w,), jnp.float32),
            pltpu.SemaphoreType.DMA,
            pltpu.SemaphoreType.DMA,
        ],
    )
    out = run(user_input.astype(jnp.int32), item_input.astype(jnp.int32),
              user_table.reshape(_NU // 2, 2 * _D),
              item_table.reshape(_NU // 2, 2 * _D),
              W.reshape(_D), jnp.broadcast_to(b.reshape(1), (_LANES,)))
    return out.reshape(_B, 1)


# speedup vs baseline: 2.0359x; 2.0359x over previous
import functools

import jax
import jax.numpy as jnp
from jax import lax
from jax.experimental import pallas as pl
from jax.experimental.pallas import tpu as pltpu
from jax.experimental.pallas import tpu_sc as plsc

_B = 16384
_D = 64
_LANES = 16
_NSLOT = 2


def _gmf_kernel(b_per_w, user_idx, item_idx, user_table,
                item_table, w_vec, bias, out_hbm,
                idx_u_v, idx_i_v, rows_u, rows_i,
                w_v, b_v, out_v, *sems):
    n_ichunks = b_per_w // 128
    n_groups = b_per_w // _LANES
    wid = lax.axis_index("s") * 2 + lax.axis_index("c")
    base = wid * b_per_w

    for j in range(n_ichunks):
        pltpu.sync_copy(user_idx.at[pl.ds(base + j * 128, 128)], idx_u_v.at[j])
        pltpu.sync_copy(item_idx.at[pl.ds(base + j * 128, 128)], idx_i_v.at[j])
    pltpu.sync_copy(w_vec, w_v)
    pltpu.sync_copy(bias, b_v)

    def issue_group(g, slot):
        pos = g * _LANES
        iu = idx_u_v[pos // 128, pl.ds(pos % 128, _LANES)]
        ii = idx_i_v[pos // 128, pl.ds(pos % 128, _LANES)]
        bu = lax.shift_right_logical(iu, 3)
        si = ii & (-8)
        for e in range(_LANES):
            pltpu.async_copy(user_table.at[bu[e]], rows_u.at[slot, e],
                             sems[slot])
            pltpu.async_copy(
                item_table.at[pl.ds(pl.multiple_of(si[e], 8), 8)],
                rows_i.at[slot, pl.ds(e * 8, 8)], sems[slot])

    def drain_group(slot):
        pltpu.make_async_copy(user_table.at[pl.ds(0, _LANES)],
                              rows_u.at[slot], sems[slot]).wait()
        pltpu.make_async_copy(item_table.at[pl.ds(0, _LANES * 8)],
                              rows_i.at[slot], sems[slot]).wait()

    w_chunks = [w_v[pl.ds(k * _LANES, _LANES)] for k in range(_D // _LANES)]
    lane = lax.iota(jnp.int32, _LANES)
    bias_vec = b_v[...]

    def compute_group(g, slot):
        pos = g * _LANES
        ru = idx_u_v[pos // 128, pl.ds(pos % 128, _LANES)] & 7
        ri = idx_i_v[pos // 128, pl.ds(pos % 128, _LANES)] & 7
        res = jnp.zeros((_LANES,), jnp.float32)
        for e in range(_LANES):
            rue = ru[e]
            rie = ri[e]
            acc = (rows_u[slot, e, rue, pl.ds(0, _LANES)]
                   * rows_i[slot, e * 8 + rie, pl.ds(0, _LANES)]
                   * w_chunks[0])
            for k in range(1, _D // _LANES):
                acc = acc + (rows_u[slot, e, rue, pl.ds(k * _LANES, _LANES)]
                             * rows_i[slot, e * 8 + rie,
                                      pl.ds(k * _LANES, _LANES)]
                             * w_chunks[k])
            res = jnp.where(lane == e, jnp.sum(acc), res)
        x = res + bias_vec
        out_v[pl.ds(pos, _LANES)] = 1.0 / (1.0 + jnp.exp(-x))

    for q in range(_NSLOT - 1):
        issue_group(q, q)

    def pipe_body(h, carry):
        g0 = h * _NSLOT
        for q in range(_NSLOT):
            g = g0 + q

            @pl.when(g + _NSLOT - 1 < n_groups)
            def _():
                issue_group(g + _NSLOT - 1, (q + _NSLOT - 1) % _NSLOT)

            drain_group(q)
            compute_group(g, q)
        return carry

    lax.fori_loop(0, n_groups // _NSLOT, pipe_body, 0)

    pltpu.sync_copy(out_v, out_hbm.at[pl.ds(base, b_per_w)])


def kernel(user_input, item_input, user_table, item_table, W, b):
    info = plsc.get_sparse_core_info()
    num_workers = info.num_cores * info.num_subcores
    b_per_w = _B // num_workers
    n_ichunks = b_per_w // 128

    mesh = plsc.VectorSubcoreMesh(core_axis_name="c", subcore_axis_name="s")
    run = pl.kernel(
        functools.partial(_gmf_kernel, b_per_w),
        mesh=mesh,
        compiler_params=pltpu.CompilerParams(needs_layout_passes=False),
        out_type=jax.ShapeDtypeStruct((_B,), jnp.float32),
        scratch_types=[
            pltpu.VMEM((n_ichunks, 128), jnp.int32),
            pltpu.VMEM((n_ichunks, 128), jnp.int32),
            pltpu.VMEM((_NSLOT, _LANES, 8, _D), jnp.float32),
            pltpu.VMEM((_NSLOT, _LANES * 8, _D), jnp.float32),
            pltpu.VMEM((_D,), jnp.float32),
            pltpu.VMEM((_LANES,), jnp.float32),
            pltpu.VMEM((b_per_w,), jnp.float32),
        ] + [pltpu.SemaphoreType.DMA] * _NSLOT,
    )
    out = run(user_input.astype(jnp.int32), item_input.astype(jnp.int32),
              user_table.reshape(-1, 8, _D), item_table,
              W.reshape(_D), jnp.broadcast_to(b.reshape(1), (_LANES,)))
    return out.reshape(_B, 1)


# speedup vs baseline: 2.2024x; 1.0818x over previous
import functools

import jax
import jax.numpy as jnp
from jax import lax
from jax.experimental import pallas as pl
from jax.experimental.pallas import tpu as pltpu
from jax.experimental.pallas import tpu_sc as plsc

_B = 16384
_D = 64
_LANES = 16
_NSLOT = 2


def _gmf_kernel(b_per_w, user_idx, item_idx, user_table,
                item_table, w_vec, bias, out_hbm,
                idx_u_v, idx_i_v, rows_u, rows_i,
                w_v, b_v, out_v, *sems):
    n_ichunks = b_per_w // 128
    n_groups = b_per_w // _LANES
    wid = lax.axis_index("s") * 2 + lax.axis_index("c")
    base = wid * b_per_w

    for j in range(n_ichunks):
        pltpu.sync_copy(user_idx.at[pl.ds(base + j * 128, 128)], idx_u_v.at[j])
        pltpu.sync_copy(item_idx.at[pl.ds(base + j * 128, 128)], idx_i_v.at[j])
    pltpu.sync_copy(w_vec, w_v)
    pltpu.sync_copy(bias, b_v)

    def issue_group(g, slot):
        pos = g * _LANES
        iu = idx_u_v[pos // 128, pl.ds(pos % 128, _LANES)]
        ii = idx_i_v[pos // 128, pl.ds(pos % 128, _LANES)]
        bu = lax.shift_right_logical(iu, 3)
        bi = lax.shift_right_logical(ii, 3)
        for e in range(_LANES):
            pltpu.async_copy(user_table.at[bu[e]], rows_u.at[slot, e],
                             sems[slot])
            pltpu.async_copy(item_table.at[bi[e]], rows_i.at[slot, e],
                             sems[slot])

    def drain_group(slot):
        pltpu.make_async_copy(user_table.at[pl.ds(0, _LANES)],
                              rows_u.at[slot], sems[slot]).wait()
        pltpu.make_async_copy(item_table.at[pl.ds(0, _LANES)],
                              rows_i.at[slot], sems[slot]).wait()

    w_chunks = [w_v[pl.ds(k * _LANES, _LANES)] for k in range(_D // _LANES)]
    lane = lax.iota(jnp.int32, _LANES)
    bias_vec = b_v[...]

    def compute_group(g, slot):
        pos = g * _LANES
        ru = idx_u_v[pos // 128, pl.ds(pos % 128, _LANES)] & 7
        ri = idx_i_v[pos // 128, pl.ds(pos % 128, _LANES)] & 7
        res = jnp.zeros((_LANES,), jnp.float32)
        for e in range(_LANES):
            rue = ru[e]
            rie = ri[e]
            acc = (rows_u[slot, e, rue, pl.ds(0, _LANES)]
                   * rows_i[slot, e, rie, pl.ds(0, _LANES)] * w_chunks[0])
            for k in range(1, _D // _LANES):
                acc = acc + (rows_u[slot, e, rue, pl.ds(k * _LANES, _LANES)]
                             * rows_i[slot, e, rie, pl.ds(k * _LANES, _LANES)]
                             * w_chunks[k])
            res = jnp.where(lane == e, jnp.sum(acc), res)
        x = res + bias_vec
        out_v[pl.ds(pos, _LANES)] = 1.0 / (1.0 + jnp.exp(-x))

    for q in range(_NSLOT - 1):
        issue_group(q, q)

    def pipe_body(h, carry):
        g0 = h * _NSLOT
        for q in range(_NSLOT):
            g = g0 + q

            @pl.when(g + _NSLOT - 1 < n_groups)
            def _():
                issue_group(g + _NSLOT - 1, (q + _NSLOT - 1) % _NSLOT)

            drain_group(q)
            compute_group(g, q)
        return carry

    lax.fori_loop(0, n_groups // _NSLOT, pipe_body, 0)

    pltpu.sync_copy(out_v, out_hbm.at[pl.ds(base, b_per_w)])


def kernel(user_input, item_input, user_table, item_table, W, b):
    info = plsc.get_sparse_core_info()
    num_workers = info.num_cores * info.num_subcores
    b_per_w = _B // num_workers
    n_ichunks = b_per_w // 128

    mesh = plsc.VectorSubcoreMesh(core_axis_name="c", subcore_axis_name="s")
    run = pl.kernel(
        functools.partial(_gmf_kernel, b_per_w),
        mesh=mesh,
        compiler_params=pltpu.CompilerParams(needs_layout_passes=False),
        out_type=jax.ShapeDtypeStruct((_B,), jnp.float32),
        scratch_types=[
            pltpu.VMEM((n_ichunks, 128), jnp.int32),
            pltpu.VMEM((n_ichunks, 128), jnp.int32),
            pltpu.VMEM((_NSLOT, _LANES, 8, _D), jnp.float32),
            pltpu.VMEM((_NSLOT, _LANES, 8, _D), jnp.float32),
            pltpu.VMEM((_D,), jnp.float32),
            pltpu.VMEM((_LANES,), jnp.float32),
            pltpu.VMEM((b_per_w,), jnp.float32),
        ] + [pltpu.SemaphoreType.DMA] * _NSLOT,
    )
    out = run(user_input.astype(jnp.int32), item_input.astype(jnp.int32),
              user_table.reshape(-1, 8, _D), item_table.reshape(-1, 8, _D),
              W.reshape(_D), jnp.broadcast_to(b.reshape(1), (_LANES,)))
    return out.reshape(_B, 1)
